# baseline (speedup 1.0000x reference)
import functools

import jax
import jax.numpy as jnp
from jax import lax
from jax.experimental import pallas as pl
from jax.experimental.pallas import tpu as pltpu
from jax.experimental.pallas import tpu_sc as plsc

NC = 2
NS = 16
NW = NC * NS
L = 16

D = 64
G = 128


def _rsqrt_newton(x):
    i = plsc.bitcast(x, jnp.int32)
    i = jnp.int32(0x5F3759DF) - (i >> 1)
    y = plsc.bitcast(i, jnp.float32)
    for _ in range(3):
        y = y * (jnp.float32(1.5) - jnp.float32(0.5) * x * y * y)
    return y


def _body(ngroups, table_hbm, idx_hbm, out_hbm, idx_v, rows_v, gsem):
    cid = lax.axis_index("c")
    sid = lax.axis_index("s")
    wid = sid * NC + cid
    rows_per_w = ngroups * G
    base = wid * rows_per_w

    pltpu.sync_copy(idx_hbm.at[wid], idx_v)

    lanes = lax.iota(jnp.int32, L)

    def group(g, carry):
        pltpu.async_copy(table_hbm.at[idx_v.at[g]], rows_v, gsem).wait()

        def block(rb, c2):
            row_ids = rb * L + lanes

            def pass1(j, accs):
                a0, a1, a2, a3 = accs
                outs = []
                for c in range(4):
                    col = jnp.full((L,), 4 * j + c, dtype=jnp.int32)
                    v = plsc.load_gather(rows_v, [row_ids, col])
                    outs.append(v * v)
                return (a0 + outs[0], a1 + outs[1], a2 + outs[2], a3 + outs[3])

            z = jnp.zeros((L,), jnp.float32)
            a0, a1, a2, a3 = lax.fori_loop(0, 16, pass1, (z, z, z, z))
            sumsq = (a0 + a1) + (a2 + a3)
            scale = jnp.where(sumsq > jnp.float32(1.0), _rsqrt_newton(sumsq),
                              jnp.float32(1.0))

            def pass2(j, c3):
                for c in range(4):
                    col = jnp.full((L,), 4 * j + c, dtype=jnp.int32)
                    v = plsc.load_gather(rows_v, [row_ids, col])
                    plsc.store_scatter(rows_v, [row_ids, col], v * scale)
                return c3

            lax.fori_loop(0, 16, pass2, c2)
            return c2

        lax.fori_loop(0, G // L, block, 0)

        pltpu.sync_copy(rows_v, out_hbm.at[pl.ds(base + g * G, G)])
        return carry

    lax.fori_loop(0, ngroups, group, 0)


@functools.partial(jax.jit, static_argnames=())
def kernel(token_ids, table):
    orig_shape = token_ids.shape
    B = token_ids.size
    assert B % (NW * G) == 0
    ngroups = B // (NW * G)
    idx = token_ids.reshape(NW, ngroups, G).astype(jnp.int32)

    mesh = plsc.VectorSubcoreMesh(
        core_axis_name="c", subcore_axis_name="s", num_cores=NC, num_subcores=NS
    )
    out = pl.kernel(
        functools.partial(_body, ngroups),
        out_type=jax.ShapeDtypeStruct((B, D), jnp.float32),
        mesh=mesh,
        scratch_types=[
            pltpu.VMEM((ngroups, G), jnp.int32),
            pltpu.VMEM((G, D), jnp.float32),
            pltpu.SemaphoreType.DMA,
        ],
        compiler_params=pltpu.CompilerParams(
            needs_layout_passes=False, use_tc_tiling_on_sc=False
        ),
    )(table, idx)
    return out.reshape(*orig_shape, D)


# speedup vs baseline: 1.0597x; 1.0597x over previous
import functools

import jax
import jax.numpy as jnp
from jax import lax
from jax.experimental import pallas as pl
from jax.experimental.pallas import tpu as pltpu
from jax.experimental.pallas import tpu_sc as plsc

NC = 2
NS = 16
NW = NC * NS
L = 16

D = 64
G = 128
NBUF = 4


def _rsqrt_newton(x):
    i = plsc.bitcast(x, jnp.int32)
    i = jnp.int32(0x5F3759DF) - (i >> 1)
    y = plsc.bitcast(i, jnp.float32)
    for _ in range(3):
        y = y * (jnp.float32(1.5) - jnp.float32(0.5) * x * y * y)
    return y


def _body(ngroups, table_hbm, idx_hbm, out_hbm, idx_v, rows_v, gsem, osem):
    cid = lax.axis_index("c")
    sid = lax.axis_index("s")
    wid = sid * NC + cid
    base = wid * ngroups * G

    pltpu.sync_copy(idx_hbm.at[wid], idx_v)

    lanes = lax.iota(jnp.int32, L)
    zero = jnp.zeros((L,), jnp.float32)
    cols = [jnp.full((L,), c, dtype=jnp.int32) for c in range(D)]

    def start_gather(g, b):
        pltpu.async_copy(table_hbm.at[idx_v.at[g]], rows_v.at[b], gsem.at[b])

    def wait_gather(b):
        pltpu.make_async_copy(
            table_hbm.at[idx_v.at[0]], rows_v.at[b], gsem.at[b]
        ).wait()

    def start_out(g, b):
        pltpu.async_copy(
            rows_v.at[b], out_hbm.at[pl.ds(base + g * G, G)], osem.at[b]
        )

    def wait_out(b):
        pltpu.make_async_copy(
            rows_v.at[b], out_hbm.at[pl.ds(base, G)], osem.at[b]
        ).wait()

    def compute(b):
        buf = rows_v.at[b]

        def block(rb, carry):
            row_ids = rb * L + lanes
            a = [zero, zero, zero, zero]
            for j in range(D):
                v = plsc.load_gather(buf, [row_ids, cols[j]])
                a[j % 4] = a[j % 4] + v * v
            sumsq = (a[0] + a[1]) + (a[2] + a[3])
            scale = jnp.where(
                sumsq > jnp.float32(1.0), _rsqrt_newton(sumsq), jnp.float32(1.0)
            )
            for j in range(D):
                v = plsc.load_gather(buf, [row_ids, cols[j]])
                plsc.store_scatter(buf, [row_ids, cols[j]], v * scale)
            return carry

        lax.fori_loop(0, G // L, block, 0)

    def group(g, b, first, prefetch):
        pb = (b - 1) % NBUF
        if not first:
            wait_out(pb)
        if prefetch:
            start_gather(g + (NBUF - 1), pb)
        wait_gather(b)
        compute(b)
        start_out(g, b)

    for b in range(NBUF - 1):
        start_gather(b, b)

    for b in range(NBUF):
        group(b, b, first=(b == 0), prefetch=True)

    @pl.loop(NBUF, ngroups - NBUF, step=NBUF)
    def steady(gbase):
        for b in range(NBUF):
            group(gbase + b, b, first=False, prefetch=True)

    for b in range(NBUF):
        group(ngroups - NBUF + b, b, first=False, prefetch=(b == 0))

    wait_out(NBUF - 1)


@functools.partial(jax.jit, static_argnames=())
def kernel(token_ids, table):
    orig_shape = token_ids.shape
    B = token_ids.size
    assert B % (NW * G) == 0
    ngroups = B // (NW * G)
    assert ngroups % NBUF == 0 and ngroups >= 2 * NBUF
    idx = token_ids.reshape(NW, ngroups, G).astype(jnp.int32)

    mesh = plsc.VectorSubcoreMesh(
        core_axis_name="c", subcore_axis_name="s", num_cores=NC, num_subcores=NS
    )
    out = pl.kernel(
        functools.partial(_body, ngroups),
        out_type=jax.ShapeDtypeStruct((B, D), jnp.float32),
        mesh=mesh,
        scratch_types=[
            pltpu.VMEM((ngroups, G), jnp.int32),
            pltpu.VMEM((NBUF, G, D), jnp.float32),
            pltpu.SemaphoreType.DMA((NBUF,)),
            pltpu.SemaphoreType.DMA((NBUF,)),
        ],
        compiler_params=pltpu.CompilerParams(
            needs_layout_passes=False, use_tc_tiling_on_sc=False
        ),
    )(table, idx)
    return out.reshape(*orig_shape, D)


# speedup vs baseline: 2.8235x; 2.6643x over previous
import functools

import jax
import jax.numpy as jnp
from jax import lax
from jax.experimental import pallas as pl
from jax.experimental.pallas import tpu as pltpu
from jax.experimental.pallas import tpu_sc as plsc

NC = 2
NS = 16
NW = NC * NS
L = 16

D = 64
G = 128
NBUF = 4


def _rsqrt_newton(x):
    i = lax.bitcast_convert_type(x, jnp.int32)
    i = jnp.int32(0x5F3759DF) - (i >> 1)
    y = lax.bitcast_convert_type(i, jnp.float32)
    for _ in range(3):
        y = y * (jnp.float32(1.5) - jnp.float32(0.5) * x * y * y)
    return y


def _body(ngroups, table_hbm, idx_hbm, out_hbm, idx_v, rows_v, gsem, osem):
    cid = lax.axis_index("c")
    sid = lax.axis_index("s")
    wid = sid * NC + cid
    base = wid * ngroups * G

    pltpu.sync_copy(idx_hbm.at[wid], idx_v)

    def start_gather(g, b):
        pltpu.async_copy(table_hbm.at[idx_v.at[g]], rows_v.at[b], gsem.at[b])

    def wait_gather(b):
        pltpu.make_async_copy(
            table_hbm.at[idx_v.at[0]], rows_v.at[b], gsem.at[b]
        ).wait()

    def start_out(g, b):
        pltpu.async_copy(
            rows_v.at[b], out_hbm.at[pl.ds(base + g * G, G)], osem.at[b]
        )

    def wait_out(b):
        pltpu.make_async_copy(
            rows_v.at[b], out_hbm.at[pl.ds(base, G)], osem.at[b]
        ).wait()

    RU = 8

    def compute(b):
        buf = rows_v.at[b]

        def quad(qb, carry):
            row0 = qb * RU
            for r in range(RU):
                row = row0 + r
                vs = [buf[row, pl.ds(c * L, L)] for c in range(D // L)]
                sq = [v * v for v in vs]
                ssq = (sq[0] + sq[1]) + (sq[2] + sq[3])
                s = jnp.sum(ssq)
                scale = jnp.where(
                    s > jnp.float32(1.0), _rsqrt_newton(s), jnp.float32(1.0)
                )
                sv = jnp.full((L,), scale, dtype=jnp.float32)
                for c in range(D // L):
                    buf[row, pl.ds(c * L, L)] = vs[c] * sv
            return carry

        lax.fori_loop(0, G // RU, quad, 0)

    def group(g, b, first, prefetch):
        pb = (b - 1) % NBUF
        if not first:
            wait_out(pb)
        if prefetch:
            start_gather(g + (NBUF - 1), pb)
        wait_gather(b)
        compute(b)
        start_out(g, b)

    for b in range(NBUF - 1):
        start_gather(b, b)

    for b in range(NBUF):
        group(b, b, first=(b == 0), prefetch=True)

    @pl.loop(NBUF, ngroups - NBUF, step=NBUF)
    def steady(gbase):
        for b in range(NBUF):
            group(gbase + b, b, first=False, prefetch=True)

    for b in range(NBUF):
        group(ngroups - NBUF + b, b, first=False, prefetch=(b == 0))

    wait_out(NBUF - 1)


@functools.partial(jax.jit, static_argnames=())
def kernel(token_ids, table):
    orig_shape = token_ids.shape
    B = token_ids.size
    assert B % (NW * G) == 0
    ngroups = B // (NW * G)
    assert ngroups % NBUF == 0 and ngroups >= 2 * NBUF
    idx = token_ids.reshape(NW, ngroups, G).astype(jnp.int32)

    mesh = plsc.VectorSubcoreMesh(
        core_axis_name="c", subcore_axis_name="s", num_cores=NC, num_subcores=NS
    )
    out = pl.kernel(
        functools.partial(_body, ngroups),
        out_type=jax.ShapeDtypeStruct((B, D), jnp.float32),
        mesh=mesh,
        scratch_types=[
            pltpu.VMEM((ngroups, G), jnp.int32),
            pltpu.VMEM((NBUF, G, D), jnp.float32),
            pltpu.SemaphoreType.DMA((NBUF,)),
            pltpu.SemaphoreType.DMA((NBUF,)),
        ],
        compiler_params=pltpu.CompilerParams(
            needs_layout_passes=False, use_tc_tiling_on_sc=False
        ),
    )(table, idx)
    return out.reshape(*orig_shape, D)


# speedup vs baseline: 3.2254x; 1.1423x over previous
import functools

import jax
import jax.numpy as jnp
from jax import lax
from jax.experimental import pallas as pl
from jax.experimental.pallas import tpu as pltpu
from jax.experimental.pallas import tpu_sc as plsc

NC = 2
NS = 16
NW = NC * NS
L = 16

D = 64
G = 128
NBUF = 4

AW = 2048
CGRP = 16


def _a_body(t_ref, o_ref):
    x = t_ref[...]
    o_ref[:, 0:64] = x.T


def _sc_gather_body(ngroups, table_hbm, idx_hbm, out_hbm, idx_v, rows_v, gsem, osem):
    cid = lax.axis_index("c")
    sid = lax.axis_index("s")
    wid = sid * NC + cid
    base = wid * ngroups * G

    pltpu.sync_copy(idx_hbm.at[wid], idx_v)

    def start_gather(g, b):
        pltpu.async_copy(table_hbm.at[idx_v.at[g]], rows_v.at[b], gsem.at[b])

    def wait_gather(b):
        pltpu.make_async_copy(
            table_hbm.at[idx_v.at[0]], rows_v.at[b], gsem.at[b]
        ).wait()

    def start_out(g, b):
        pltpu.async_copy(
            rows_v.at[b], out_hbm.at[pl.ds(base + g * G, G)], osem.at[b]
        )

    def wait_out(b):
        pltpu.make_async_copy(
            rows_v.at[b], out_hbm.at[pl.ds(base, G)], osem.at[b]
        ).wait()

    def group(g, b, first, prefetch):
        pb = (b - 1) % NBUF
        if not first:
            wait_out(pb)
        if prefetch:
            start_gather(g + (NBUF - 1), pb)
        wait_gather(b)
        start_out(g, b)

    for b in range(NBUF - 1):
        start_gather(b, b)

    for b in range(NBUF):
        group(b, b, first=(b == 0), prefetch=True)

    @pl.loop(NBUF, ngroups - NBUF, step=NBUF)
    def steady(gbase):
        for b in range(NBUF):
            group(gbase + b, b, first=False, prefetch=True)

    for b in range(NBUF):
        group(ngroups - NBUF + b, b, first=False, prefetch=(b == 0))

    wait_out(NBUF - 1)


def _c_body(x_ref, o_ref):
    x = x_ref[...]
    half = CGRP * G // 2

    def scale_of(h):
        ssq = jnp.sum(h * h, axis=1, keepdims=True)
        return jnp.minimum(
            jnp.float32(1.0), lax.rsqrt(jnp.maximum(ssq, jnp.float32(1e-14)))
        )

    e = x[:, 0:D]
    o = x[:, D : 2 * D]
    sc = jnp.concatenate(
        [
            jnp.broadcast_to(scale_of(e), (half, D)),
            jnp.broadcast_to(scale_of(o), (half, D)),
        ],
        axis=1,
    )
    xt = (x * sc).T
    o_ref[0, :, :] = jnp.concatenate([xt[0:D, :], xt[D : 2 * D, :]], axis=1)


@functools.partial(jax.jit, static_argnames=())
def kernel(token_ids, table):
    V = table.shape[0]
    S, P = token_ids.shape
    B = S * P
    assert B % (NW * G) == 0
    ngroups = B // (NW * G)
    assert ngroups % NBUF == 0 and ngroups >= 2 * NBUF

    na = pl.cdiv(V, AW)
    a_out = pl.pallas_call(
        _a_body,
        grid=(na,),
        in_specs=[pl.BlockSpec((64, AW), lambda i: (0, i))],
        out_specs=pl.BlockSpec((AW, 128), lambda i: (i, 0)),
        out_shape=jax.ShapeDtypeStruct((na * AW, 128), jnp.float32),
    )(table.T)
    table_lin = a_out.reshape(-1).reshape(2 * na * AW, D)

    nblk = B // (CGRP * G)
    idx = (
        token_ids.T.reshape(nblk, 2, CGRP * G // 2)
        .transpose(0, 2, 1)
        .reshape(NW, ngroups, G)
        .astype(jnp.int32)
        * 2
    )

    mesh = plsc.VectorSubcoreMesh(
        core_axis_name="c", subcore_axis_name="s", num_cores=NC, num_subcores=NS
    )
    gathered = pl.kernel(
        functools.partial(_sc_gather_body, ngroups),
        out_type=jax.ShapeDtypeStruct((B, D), jnp.float32),
        mesh=mesh,
        scratch_types=[
            pltpu.VMEM((ngroups, G), jnp.int32),
            pltpu.VMEM((NBUF, G, D), jnp.float32),
            pltpu.SemaphoreType.DMA((NBUF,)),
            pltpu.SemaphoreType.DMA((NBUF,)),
        ],
        compiler_params=pltpu.CompilerParams(
            needs_layout_passes=False, use_tc_tiling_on_sc=False
        ),
    )(table_lin, idx)

    c_in = gathered.reshape(-1).reshape(B // 2, 128)
    nc = B // (CGRP * G)
    steps_per_pos = (S // G) // CGRP
    c_out = pl.pallas_call(
        _c_body,
        grid=(nc,),
        in_specs=[pl.BlockSpec((CGRP * 64, 128), lambda k: (k, 0))],
        out_specs=pl.BlockSpec(
            (1, D, CGRP * G), lambda k: (k // steps_per_pos, 0, k % steps_per_pos)
        ),
        out_shape=jax.ShapeDtypeStruct((P, D, S), jnp.float32),
    )(c_in)
    return jnp.transpose(c_out, (2, 0, 1))
